# Initial kernel scaffold; baseline (speedup 1.0000x reference)
import jax
import jax.numpy as jnp
from jax.experimental import pallas as pl


def kernel(messages, probs):
    raise NotImplementedError("write your pallas kernel here")


# speedup vs baseline: 1.2154x; 1.2154x over previous
import functools

import numpy as np
import jax
import jax.numpy as jnp
from jax import lax
from jax.experimental import pallas as pl
from jax.experimental.pallas import tpu as pltpu
from jax.experimental.pallas import tpu_sc as plsc

B, L, V = 4096, 20, 64
P = 0.1
NWORKERS = 32
ROWS = B * L
ROWS_PER_W = ROWS // NWORKERS
CHUNK = 128
CHUNKS_PER_W = ROWS_PER_W // CHUNK
FIRE = 4
OUTER = CHUNKS_PER_W // FIRE
FILL_CHUNKS = 3


def _threefry_uniform_mask():
    def rotl(x, d):
        return (x << np.uint32(d)) | (x >> np.uint32(32 - d))

    n = B * L
    i = np.arange(n, dtype=np.uint64)
    x0 = (i >> np.uint64(32)).astype(np.uint32)
    x1 = (i & np.uint64(0xFFFFFFFF)).astype(np.uint32)
    ks0, ks1 = np.uint32(0), np.uint32(42)
    ks2 = ks0 ^ ks1 ^ np.uint32(0x1BD11BDA)
    x0 = (x0 + ks0).astype(np.uint32)
    x1 = (x1 + ks1).astype(np.uint32)
    rots = ((13, 15, 26, 6), (17, 29, 16, 24))
    keys = [(ks1, ks2), (ks2, ks0), (ks0, ks1), (ks1, ks2), (ks2, ks0)]
    for r in range(5):
        for d in rots[r % 2]:
            x0 = (x0 + x1).astype(np.uint32)
            x1 = rotl(x1, d) ^ x0
        x0 = (x0 + keys[r][0]).astype(np.uint32)
        x1 = (x1 + keys[r][1] + np.uint32(r + 1)).astype(np.uint32)
    bits = x0 ^ x1
    flo = ((bits >> np.uint32(9)) | np.uint32(0x3F800000)).view(np.float32)
    flo = np.maximum(np.float32(0.0), flo - np.float32(1.0))
    return (flo < np.float32(P)).reshape(B, L)


def _precompute():
    mask = _threefry_uniform_mask()
    src = np.argsort(mask, axis=1, kind="stable")
    flat_src = (src + np.arange(B)[:, None] * L).reshape(-1).astype(np.int32)
    src_idx = flat_src.reshape(NWORKERS, CHUNKS_PER_W, CHUNK)

    nkeep = (~mask).sum(axis=1)
    fill = np.arange(L)[None, :] >= nkeep[:, None]
    fill_rows = np.nonzero(fill.reshape(-1))[0].astype(np.int32)
    fill_idx = np.zeros((NWORKERS, FILL_CHUNKS, CHUNK), np.int32)
    for t in range(NWORKERS):
        mine = fill_rows[(fill_rows >= t * ROWS_PER_W)
                         & (fill_rows < (t + 1) * ROWS_PER_W)]
        assert 1 <= mine.size <= FILL_CHUNKS * CHUNK
        padded = np.full(FILL_CHUNKS * CHUNK, mine[0], np.int32)
        padded[:mine.size] = mine
        fill_idx[t] = padded.reshape(FILL_CHUNKS, CHUNK)
    return src_idx, fill_idx


_SRC_IDX, _FILL_IDX = _precompute()

_sc_mesh = plsc.VectorSubcoreMesh(core_axis_name="c", subcore_axis_name="s")


@functools.partial(
    pl.kernel,
    mesh=_sc_mesh,
    out_type=jax.ShapeDtypeStruct((ROWS, V), jnp.float32),
    compiler_params=pltpu.CompilerParams(use_tc_tiling_on_sc=False),
    scratch_types=[
        pltpu.VMEM((CHUNKS_PER_W, CHUNK), jnp.int32),
        pltpu.VMEM((FILL_CHUNKS, CHUNK), jnp.int32),
        pltpu.VMEM((FIRE * CHUNK, V), jnp.float32),
        pltpu.VMEM((CHUNK, V), jnp.float32),
        pltpu.SemaphoreType.DMA,
    ],
)
def _sc_deletion(msg_hbm, src_hbm, fill_hbm, out_hbm,
                 idx_v, fill_v, gbuf, e0, sem):
    wid = lax.axis_index("s") * 2 + lax.axis_index("c")
    base = wid * ROWS_PER_W
    pltpu.sync_copy(src_hbm.at[wid], idx_v)
    pltpu.sync_copy(fill_hbm.at[wid], fill_v)

    onehot16 = jnp.where(jnp.arange(16, dtype=jnp.int32) == 0,
                         jnp.float32(1.0), jnp.float32(0.0))
    zeros16 = jnp.zeros((16,), jnp.float32)

    def _mk_onehot(i, carry):
        e0[i, pl.ds(0, 16)] = onehot16
        e0[i, pl.ds(16, 16)] = zeros16
        e0[i, pl.ds(32, 16)] = zeros16
        e0[i, pl.ds(48, 16)] = zeros16
        return carry

    lax.fori_loop(0, CHUNK, _mk_onehot, 0)

    for i in range(OUTER):
        descs = []
        for j in range(FIRE):
            c = i * FIRE + j
            descs.append(pltpu.async_copy(
                msg_hbm.at[idx_v.at[c]],
                gbuf.at[pl.ds(j * CHUNK, CHUNK)], sem))
        for d in descs:
            d.wait()
        pltpu.sync_copy(gbuf, out_hbm.at[pl.ds(base + i * FIRE * CHUNK,
                                               FIRE * CHUNK)])

    for c in range(FILL_CHUNKS):
        pltpu.async_copy(e0, out_hbm.at[fill_v.at[c]], sem).wait()


def _probs_body(p_ref, o_ref):
    x = p_ref[...]
    col = lax.broadcasted_iota(jnp.int32, x.shape, 1)
    tail = jnp.where(col == 0, jnp.float32(0.0), x * jnp.float32(1.0 - P))
    head = jnp.float32(1.0) - jnp.sum(tail, axis=-1, keepdims=True)
    o_ref[...] = jnp.where(col == 0, head, tail)


_probs_tc = pl.pallas_call(
    _probs_body,
    grid=(CHUNKS_PER_W,),
    in_specs=[pl.BlockSpec((B, V), lambda i: (i, 0))],
    out_specs=pl.BlockSpec((B, V), lambda i: (i, 0)),
    out_shape=jax.ShapeDtypeStruct((ROWS, V), jnp.float32),
)


def kernel(messages, probs):
    msg_flat = messages.reshape(ROWS, V)
    noisy_m = _sc_deletion(msg_flat, _SRC_IDX, _FILL_IDX).reshape(B, L, V)
    noisy_p = _probs_tc(probs.reshape(ROWS, V)).reshape(B, L, V)
    return (noisy_m, noisy_p, messages, probs)
